# baseline (speedup 1.0000x reference)
import functools

import jax
import jax.numpy as jnp
from jax.experimental import pallas as pl
from jax.experimental.pallas import tpu as pltpu

_RATIO = 0.25
_K = 32
_QT = 8
_LANES = 128
_BIG = 2**30


def _mm_body(xp_ref, w_ref, g_ref):
    g_ref[...] = jnp.dot(xp_ref[...], w_ref[...],
                         preferred_element_type=jnp.float32)


def _g_matmul(xp, W):
    n, d = xp.shape
    dout = W.shape[1]
    rows = 2000
    return pl.pallas_call(
        _mm_body,
        grid=(n // rows,),
        in_specs=[pl.BlockSpec((rows, d), lambda i: (i, 0)),
                  pl.BlockSpec((d, dout), lambda i: (0, 0))],
        out_specs=pl.BlockSpec((rows, dout), lambda i: (i, 0)),
        out_shape=jax.ShapeDtypeStruct((n, dout), jnp.float32),
    )(xp, W)


def _knn_body(bounds_ref, pos_t_ref, posq_ref, qs_ref, qe_ref, col_ref,
              d2_ref, *, n):
    i = pl.program_id(0)
    clo = bounds_ref[i, 0]
    chi = bounds_ref[i, 1]

    qx = posq_ref[:, 0:1]
    qy = posq_ref[:, 1:2]
    qz = posq_ref[:, 2:3]
    qs = qs_ref[...]
    qe = qe_ref[...]
    iota = jax.lax.broadcasted_iota(jnp.int32, (_QT, _LANES), 1)
    inf = jnp.float32(jnp.inf)

    def fill(c, _):
        o = pl.multiple_of(c * _LANES, _LANES)
        px = pos_t_ref[0:1, pl.ds(o, _LANES)]
        py = pos_t_ref[1:2, pl.ds(o, _LANES)]
        pz = pos_t_ref[2:3, pl.ds(o, _LANES)]
        dx = qx - px
        dy = qy - py
        dz = qz - pz
        d2 = dx * dx + dy * dy + dz * dz
        gidx = iota + c * _LANES
        ok = (gidx >= qs) & (gidx < qe)
        d2_ref[:, pl.ds(o, _LANES)] = jnp.where(ok, d2, inf)
        return 0

    jax.lax.fori_loop(clo, chi, fill, 0)

    lv = jnp.full((_QT, 1), -jnp.inf, jnp.float32)
    li = jnp.full((_QT, 1), -1, jnp.int32)
    for k in range(_K):
        def scan(c, carry):
            mv, mi = carry
            o = pl.multiple_of(c * _LANES, _LANES)
            d2 = d2_ref[:, pl.ds(o, _LANES)]
            gidx = iota + c * _LANES
            elig = (d2 > lv) | ((d2 == lv) & (gidx > li))
            t = jnp.where(elig, d2, inf)
            take = t < mv
            mi = jnp.where(take, gidx, mi)
            mv = jnp.where(take, t, mv)
            return mv, mi

        mv0 = jnp.full((_QT, _LANES), jnp.inf, jnp.float32)
        mi0 = jnp.full((_QT, _LANES), _BIG, jnp.int32)
        mv, mi = jax.lax.fori_loop(clo, chi, scan, (mv0, mi0))
        m = jnp.min(mv, axis=1, keepdims=True)
        bi = jnp.min(jnp.where(mv == m, mi, _BIG), axis=1, keepdims=True)
        col_ref[:, k:k + 1] = jnp.minimum(bi, n - 1)
        lv, li = m, bi


def _knn_pallas(pos_t, pos_qp, qs, qe, bounds, n):
    nqp = pos_qp.shape[0]
    npad = pos_t.shape[1]
    nsteps = nqp // _QT
    return pl.pallas_call(
        functools.partial(_knn_body, n=n),
        grid=(nsteps,),
        in_specs=[
            pl.BlockSpec(memory_space=pltpu.SMEM),
            pl.BlockSpec((3, npad), lambda i: (0, 0)),
            pl.BlockSpec((_QT, 3), lambda i: (i, 0)),
            pl.BlockSpec((_QT, 1), lambda i: (i, 0)),
            pl.BlockSpec((_QT, 1), lambda i: (i, 0)),
        ],
        out_specs=pl.BlockSpec((_QT, _K), lambda i: (i, 0)),
        out_shape=jax.ShapeDtypeStruct((nqp, _K), jnp.int32),
        scratch_shapes=[pltpu.VMEM((_QT, npad), jnp.float32)],
    )(bounds, pos_t, pos_qp, qs, qe)


def kernel(x, pos, batch, W, b):
    n, d = x.shape
    num_idxs = int(n * _RATIO)
    perm = jax.random.permutation(jax.random.key(42), n)[:num_idxs]
    idx = jnp.sort(perm)
    pos_q = jnp.take(pos, idx, axis=0)
    batch_q = jnp.take(batch, idx, axis=0)

    nqp = ((num_idxs + 4 * _QT - 1) // (4 * _QT)) * (4 * _QT)
    qs = jnp.searchsorted(batch, batch_q, side="left").astype(jnp.int32)
    qe = jnp.searchsorted(batch, batch_q, side="right").astype(jnp.int32)
    pad = nqp - num_idxs
    qs_p = jnp.pad(qs, (0, pad))
    qe_p = jnp.pad(qe, (0, pad))
    pos_qp = jnp.pad(pos_q, ((0, pad), (0, 0)))

    npad = ((n + _LANES - 1) // _LANES) * _LANES
    pos_t = jnp.pad(pos.T, ((0, 0), (0, npad - n)))

    nsteps = nqp // _QT
    qs_t = qs_p.reshape(nsteps, _QT)
    qe_t = qe_p.reshape(nsteps, _QT)
    bounds = jnp.stack(
        [qs_t.min(axis=1) // _LANES,
         (qe_t.max(axis=1) + _LANES - 1) // _LANES], axis=1).astype(jnp.int32)

    col = _knn_pallas(pos_t, pos_qp, qs_p[:, None], qe_p[:, None], bounds, n)
    col = col[:num_idxs]

    g = _g_matmul(jnp.concatenate([x, pos], axis=1), W)
    gmax = jnp.max(jnp.take(g, col, axis=0, mode="clip"), axis=1)
    c = pos_q @ W[d:]
    out = jax.nn.relu(gmax - c + b)
    return (out, pos_q, batch_q)


# speedup vs baseline: 1.6439x; 1.6439x over previous
import functools

import jax
import jax.numpy as jnp
from jax.experimental import pallas as pl
from jax.experimental.pallas import tpu as pltpu

_RATIO = 0.25
_K = 32
_QT = 8
_LANES = 128
_W = 512
_BIG = 2**30


def _mm_body(xp_ref, w_ref, g_ref):
    g_ref[...] = jnp.dot(xp_ref[...], w_ref[...],
                         preferred_element_type=jnp.float32)


def _g_matmul(xp, W):
    n, d = xp.shape
    dout = W.shape[1]
    rows = 2000
    return pl.pallas_call(
        _mm_body,
        grid=(n // rows,),
        in_specs=[pl.BlockSpec((rows, d), lambda i: (i, 0)),
                  pl.BlockSpec((d, dout), lambda i: (0, 0))],
        out_specs=pl.BlockSpec((rows, dout), lambda i: (i, 0)),
        out_shape=jax.ShapeDtypeStruct((n, dout), jnp.float32),
    )(xp, W)


def _knn_body(bounds_ref, pos_t_ref, posq_ref, qs_ref, qe_ref, col_ref,
              d2_ref, *, n):
    i = pl.program_id(0)
    clo = bounds_ref[i, 0]
    chi = bounds_ref[i, 1]

    qx = posq_ref[:, 0:1]
    qy = posq_ref[:, 1:2]
    qz = posq_ref[:, 2:3]
    qs = qs_ref[...]
    qe = qe_ref[...]
    iota = jax.lax.broadcasted_iota(jnp.int32, (_QT, _W), 1)
    inf = jnp.float32(jnp.inf)

    def fill(c, _):
        o = pl.multiple_of(c * _W, _W)
        px = pos_t_ref[0:1, pl.ds(o, _W)]
        py = pos_t_ref[1:2, pl.ds(o, _W)]
        pz = pos_t_ref[2:3, pl.ds(o, _W)]
        dx = qx - px
        dy = qy - py
        dz = qz - pz
        d2 = dx * dx + dy * dy + dz * dz
        gidx = iota + c * _W
        ok = (gidx >= qs) & (gidx < qe)
        d2_ref[:, pl.ds(o, _W)] = jnp.where(ok, d2, inf)
        return 0

    jax.lax.fori_loop(clo, chi, fill, 0)

    lv = jnp.full((_QT, 1), -jnp.inf, jnp.float32)
    li = jnp.full((_QT, 1), -1, jnp.int32)
    for k in range(_K):
        def scan(c, carry):
            mv, mi = carry
            o = pl.multiple_of(c * _W, _W)
            d2 = d2_ref[:, pl.ds(o, _W)]
            gidx = iota + c * _W
            elig = (d2 > lv) | ((d2 == lv) & (gidx > li))
            take = elig & (d2 < mv)
            mi = jnp.where(take, gidx, mi)
            mv = jnp.where(take, d2, mv)
            return mv, mi

        mv0 = jnp.full((_QT, _W), jnp.inf, jnp.float32)
        mi0 = jnp.full((_QT, _W), _BIG, jnp.int32)
        mv, mi = jax.lax.fori_loop(clo, chi, scan, (mv0, mi0))
        m = jnp.min(mv, axis=1, keepdims=True)
        bi = jnp.min(jnp.where(mv == m, mi, _BIG), axis=1, keepdims=True)
        col_ref[:, k:k + 1] = jnp.minimum(bi, n - 1)
        lv, li = m, bi


def _knn_pallas(pos_t, pos_qp, qs, qe, bounds, n):
    nqp = pos_qp.shape[0]
    npad = pos_t.shape[1]
    nsteps = nqp // _QT
    return pl.pallas_call(
        functools.partial(_knn_body, n=n),
        grid=(nsteps,),
        in_specs=[
            pl.BlockSpec(memory_space=pltpu.SMEM),
            pl.BlockSpec((3, npad), lambda i: (0, 0)),
            pl.BlockSpec((_QT, 3), lambda i: (i, 0)),
            pl.BlockSpec((_QT, 1), lambda i: (i, 0)),
            pl.BlockSpec((_QT, 1), lambda i: (i, 0)),
        ],
        out_specs=pl.BlockSpec((_QT, _K), lambda i: (i, 0)),
        out_shape=jax.ShapeDtypeStruct((nqp, _K), jnp.int32),
        scratch_shapes=[pltpu.VMEM((_QT, npad), jnp.float32)],
    )(bounds, pos_t, pos_qp, qs, qe)


def kernel(x, pos, batch, W, b):
    n, d = x.shape
    num_idxs = int(n * _RATIO)
    perm = jax.random.permutation(jax.random.key(42), n)[:num_idxs]
    idx = jnp.sort(perm)
    pos_q = jnp.take(pos, idx, axis=0)
    batch_q = jnp.take(batch, idx, axis=0)

    nqp = ((num_idxs + 4 * _QT - 1) // (4 * _QT)) * (4 * _QT)
    qs = jnp.searchsorted(batch, batch_q, side="left").astype(jnp.int32)
    qe = jnp.searchsorted(batch, batch_q, side="right").astype(jnp.int32)
    pad = nqp - num_idxs
    qs_p = jnp.pad(qs, (0, pad))
    qe_p = jnp.pad(qe, (0, pad))
    pos_qp = jnp.pad(pos_q, ((0, pad), (0, 0)))

    npad = ((n + _W - 1) // _W) * _W
    pos_t = jnp.pad(pos.T, ((0, 0), (0, npad - n)))

    nsteps = nqp // _QT
    qs_t = qs_p.reshape(nsteps, _QT)
    qe_t = qe_p.reshape(nsteps, _QT)
    bounds = jnp.stack(
        [qs_t.min(axis=1) // _W,
         (qe_t.max(axis=1) + _W - 1) // _W], axis=1).astype(jnp.int32)

    col = _knn_pallas(pos_t, pos_qp, qs_p[:, None], qe_p[:, None], bounds, n)
    col = col[:num_idxs]

    g = _g_matmul(jnp.concatenate([x, pos], axis=1), W)
    gmax = jnp.max(jnp.take(g, col, axis=0, mode="clip"), axis=1)
    c = pos_q @ W[d:]
    out = jax.nn.relu(gmax - c + b)
    return (out, pos_q, batch_q)


# speedup vs baseline: 1.6823x; 1.0234x over previous
import functools

import jax
import jax.numpy as jnp
from jax import lax
from jax.experimental import pallas as pl
from jax.experimental.pallas import tpu as pltpu
from jax.experimental.pallas import tpu_sc as plsc

_RATIO = 0.25
_K = 32
_QT = 8
_LANES = 128
_W = 512
_BIG = 2**30


def _mm_body(xp_ref, w_ref, g_ref):
    g_ref[...] = jnp.dot(xp_ref[...], w_ref[...],
                         preferred_element_type=jnp.float32)


def _g_matmul(xp, W, rows=2000):
    n, d = xp.shape
    dout = W.shape[1]
    return pl.pallas_call(
        _mm_body,
        grid=(n // rows,),
        in_specs=[pl.BlockSpec((rows, d), lambda i: (i, 0)),
                  pl.BlockSpec((d, dout), lambda i: (0, 0))],
        out_specs=pl.BlockSpec((rows, dout), lambda i: (i, 0)),
        out_shape=jax.ShapeDtypeStruct((n, dout), jnp.float32),
    )(xp, W)


def _knn_body(bounds_ref, pos_t_ref, posq_ref, qs_ref, qe_ref, col_ref,
              d2_ref, *, n):
    i = pl.program_id(0)
    clo = bounds_ref[i, 0]
    chi = bounds_ref[i, 1]

    qx = posq_ref[:, 0:1]
    qy = posq_ref[:, 1:2]
    qz = posq_ref[:, 2:3]
    qs = qs_ref[...]
    qe = qe_ref[...]
    iota = jax.lax.broadcasted_iota(jnp.int32, (_QT, _W), 1)
    inf = jnp.float32(jnp.inf)

    def fill(c, _):
        o = pl.multiple_of(c * _W, _W)
        px = pos_t_ref[0:1, pl.ds(o, _W)]
        py = pos_t_ref[1:2, pl.ds(o, _W)]
        pz = pos_t_ref[2:3, pl.ds(o, _W)]
        dx = qx - px
        dy = qy - py
        dz = qz - pz
        d2 = dx * dx + dy * dy + dz * dz
        gidx = iota + c * _W
        ok = (gidx >= qs) & (gidx < qe)
        d2_ref[:, pl.ds(o, _W)] = jnp.where(ok, d2, inf)
        return 0

    jax.lax.fori_loop(clo, chi, fill, 0)

    lv = jnp.full((_QT, 1), -jnp.inf, jnp.float32)
    li = jnp.full((_QT, 1), -1, jnp.int32)
    for k in range(_K):
        def scan(c, carry):
            mv, mi = carry
            o = pl.multiple_of(c * _W, _W)
            d2 = d2_ref[:, pl.ds(o, _W)]
            gidx = iota + c * _W
            elig = (d2 > lv) | ((d2 == lv) & (gidx > li))
            take = elig & (d2 < mv)
            mi = jnp.where(take, gidx, mi)
            mv = jnp.where(take, d2, mv)
            return mv, mi

        mv0 = jnp.full((_QT, _W), jnp.inf, jnp.float32)
        mi0 = jnp.full((_QT, _W), _BIG, jnp.int32)
        mv, mi = jax.lax.fori_loop(clo, chi, scan, (mv0, mi0))
        m = jnp.min(mv, axis=1, keepdims=True)
        bi = jnp.min(jnp.where(mv == m, mi, _BIG), axis=1, keepdims=True)
        col_ref[:, k:k + 1] = jnp.minimum(bi, n - 1)
        lv, li = m, bi


def _knn_pallas(pos_t, pos_qp, qs, qe, bounds, n):
    nqp = pos_qp.shape[0]
    npad = pos_t.shape[1]
    nsteps = nqp // _QT
    return pl.pallas_call(
        functools.partial(_knn_body, n=n),
        grid=(nsteps,),
        in_specs=[
            pl.BlockSpec(memory_space=pltpu.SMEM),
            pl.BlockSpec((3, npad), lambda i: (0, 0)),
            pl.BlockSpec((_QT, 3), lambda i: (i, 0)),
            pl.BlockSpec((_QT, 1), lambda i: (i, 0)),
            pl.BlockSpec((_QT, 1), lambda i: (i, 0)),
        ],
        out_specs=pl.BlockSpec((_QT, _K), lambda i: (i, 0)),
        out_shape=jax.ShapeDtypeStruct((nqp, _K), jnp.int32),
        scratch_shapes=[pltpu.VMEM((_QT, npad), jnp.float32)],
    )(bounds, pos_t, pos_qp, qs, qe)


def _ld16(ref2d, r, c):
    return ref2d[r, pl.ds(c, 16)]


def _conv_sc(g, col_flat, adj_flat, nqp, dout):
    nw = 32
    bq = 4
    nq_w = nqp // nw
    nb = nq_w // bq
    mesh = plsc.VectorSubcoreMesh(core_axis_name="c", subcore_axis_name="s")

    @functools.partial(
        pl.kernel, mesh=mesh,
        out_type=jax.ShapeDtypeStruct((nqp * dout,), jnp.float32),
        scratch_types=[
            pltpu.VMEM((bq * _K,), jnp.int32),
            pltpu.VMEM((bq * _K, dout), jnp.float32),
            pltpu.VMEM((bq * dout,), jnp.float32),
            pltpu.VMEM((bq * dout,), jnp.float32),
            pltpu.SemaphoreType.DMA,
        ])
    def conv(g_hbm, colf_hbm, adjf_hbm, outf_hbm, idx_v, rows_v, adj_v,
             outb_v, sem):
        wid = lax.axis_index("s") * 2 + lax.axis_index("c")
        base_q = wid * nq_w

        def batch_body(bi, _):
            q0 = base_q + bi * bq
            pltpu.sync_copy(colf_hbm.at[pl.ds(q0 * _K, bq * _K)], idx_v)
            pltpu.async_copy(g_hbm.at[idx_v], rows_v, sem).wait()
            pltpu.sync_copy(adjf_hbm.at[pl.ds(q0 * dout, bq * dout)], adj_v)
            for q in range(bq):
                neg = jnp.full((16,), -jnp.inf, jnp.float32)

                def red(j, accs):
                    return tuple(
                        jnp.maximum(accs[t], _ld16(rows_v, q * _K + j, 16 * t))
                        for t in range(dout // 16))

                accs = lax.fori_loop(0, _K, red, (neg,) * (dout // 16))
                for t in range(dout // 16):
                    a = adj_v[pl.ds(q * dout + 16 * t, 16)]
                    outb_v[pl.ds(q * dout + 16 * t, 16)] = (
                        jnp.maximum(accs[t] + a, 0.0))
            pltpu.sync_copy(outb_v,
                            outf_hbm.at[pl.ds(q0 * dout, bq * dout)])
            return 0

        lax.fori_loop(0, nb, batch_body, 0)

    return conv(g, col_flat, adj_flat).reshape(nqp, dout)


def kernel(x, pos, batch, W, b):
    n, d = x.shape
    num_idxs = int(n * _RATIO)
    perm = jax.random.permutation(jax.random.key(42), n)[:num_idxs]
    idx = jnp.sort(perm)
    pos_q = jnp.take(pos, idx, axis=0)
    batch_q = jnp.take(batch, idx, axis=0)

    nqp = ((num_idxs + 255) // 256) * 256
    qs = jnp.searchsorted(batch, batch_q, side="left").astype(jnp.int32)
    qe = jnp.searchsorted(batch, batch_q, side="right").astype(jnp.int32)
    pad = nqp - num_idxs
    qs_p = jnp.pad(qs, (0, pad))
    qe_p = jnp.pad(qe, (0, pad))
    pos_qp = jnp.pad(pos_q, ((0, pad), (0, 0)))

    npad = ((n + _W - 1) // _W) * _W
    pos_t = jnp.pad(pos.T, ((0, 0), (0, npad - n)))

    nsteps = nqp // _QT
    qs_t = qs_p.reshape(nsteps, _QT)
    qe_t = qe_p.reshape(nsteps, _QT)
    bounds = jnp.stack(
        [qs_t.min(axis=1) // _W,
         (qe_t.max(axis=1) + _W - 1) // _W], axis=1).astype(jnp.int32)

    col = _knn_pallas(pos_t, pos_qp, qs_p[:, None], qe_p[:, None], bounds, n)

    g = _g_matmul(jnp.concatenate([x, pos], axis=1), W)
    adj = b[None, :] - _g_matmul(pos_qp, W[d:], rows=1568)
    out_p = _conv_sc(g, col.reshape(-1), adj.reshape(-1), nqp, W.shape[1])
    out = out_p[:num_idxs]
    return (out, pos_q, batch_q)
